# baseline (speedup 1.0000x reference)
import functools

import jax
import jax.numpy as jnp
import numpy as np
from jax import lax
from jax.experimental import pallas as pl
from jax.experimental.pallas import tpu as pltpu
from jax.experimental.pallas import tpu_sc as plsc

B = 8
C = 19
P = 512 * 512
N = B * P
KK = 100000
THRESH = 0.7
THRESH_BITS = int(np.float32(THRESH).view(np.int32))
ONE_BITS = int(np.float32(1.0).view(np.int32))

TL = 4096
NG = P // (8 * TL)

SPLIT = 7
NSC = (B - SPLIT) * P
NW = 32
WPIX = NSC // NW
SCQ = 4096


def _ce_pg(x, t):
    m = jnp.max(x, axis=0)
    e = jnp.exp(x - m[None])
    s = jnp.sum(e, axis=0)
    iota = lax.broadcasted_iota(jnp.int32, (C, 8, TL), 0)
    onehot = (iota == t[None]).astype(jnp.float32)
    st = jnp.sum(x * onehot, axis=0)
    ce = (m + jnp.log(s)) - st
    pg = jnp.exp(st - m) / s
    return ce, pg


def _fused_kernel(nb, score_ref, target_ref, s7_ref, c7_ref, acc_s, acc_c):
    b = pl.program_id(0)
    g = pl.program_id(1)

    @pl.when((b == 0) & (g == 0))
    def _init():
        acc_s[...] = jnp.zeros((8, TL), jnp.float32)
        acc_c[...] = jnp.zeros((8, TL), jnp.float32)

    ce, pg = _ce_pg(score_ref[0, :, 0], target_ref[0, 0, 0])
    keep = (pg < THRESH).astype(jnp.float32)
    acc_s[...] += ce * keep
    acc_c[...] += keep

    @pl.when((b == nb - 1) & (g == NG - 1))
    def _finish():
        s7_ref[...] = jnp.sum(acc_s[...])[None, None]
        c7_ref[...] = jnp.sum(acc_c[...])[None, None]


def _fold_kernel(s_ref, stm_ref, s7_ref, c7_ref, acc_s, acc_c):
    g = pl.program_id(0)

    @pl.when(g == 0)
    def _init():
        acc_s[...] = jnp.zeros((8, TL), jnp.float32)
        acc_c[...] = jnp.zeros((8, TL), jnp.float32)

    s = s_ref[0]
    stm = stm_ref[0]
    ce = jnp.log(s) - stm
    keep = (jnp.exp(stm) < jnp.float32(THRESH) * s).astype(jnp.float32)
    acc_s[...] += ce * keep
    acc_c[...] += keep

    ngf = NSC // (8 * TL)

    @pl.when(g == ngf - 1)
    def _finish():
        s7_ref[...] = jnp.sum(acc_s[...])[None, None]
        c7_ref[...] = jnp.sum(acc_c[...])[None, None]


def _sc_body(score_ref, target_ref, s_out, stm_out, xbuf, tbuf, sbuf, stmbuf):
    cid = lax.axis_index("c")
    sid = lax.axis_index("s")
    wid = sid * 2 + cid
    base = wid * WPIX

    for k in range(WPIX // SCQ):
        cbase = base + k * SCQ
        for c in range(C):
            pltpu.sync_copy(score_ref.at[pl.ds(c * NSC + cbase, SCQ)],
                            xbuf.at[pl.ds(c * SCQ, SCQ)])
        pltpu.sync_copy(target_ref.at[pl.ds(cbase, SCQ)], tbuf)

        def body(i, carry):
            off = i * 16
            sl = pl.ds(off, 16)
            t = tbuf[sl]
            xs = [xbuf[pl.ds(c * SCQ + off, 16)] for c in range(C)]
            m = xs[0]
            for c in range(1, C):
                m = jnp.maximum(m, xs[c])
            st = jnp.where(t == 0, xs[0], jnp.zeros((16,), jnp.float32))
            for c in range(1, C):
                st = jnp.where(t == c, xs[c], st)
            s = jnp.exp(xs[0] - m)
            for c in range(1, C):
                s = s + jnp.exp(xs[c] - m)
            sbuf[sl] = s
            stmbuf[sl] = st - m
            return carry

        lax.fori_loop(0, SCQ // 16, body, jnp.int32(0))
        pltpu.sync_copy(sbuf, s_out.at[pl.ds(cbase, SCQ)])
        pltpu.sync_copy(stmbuf, stm_out.at[pl.ds(cbase, SCQ)])


def _sc_stage(score_tail, target_tail):
    mesh = plsc.VectorSubcoreMesh(core_axis_name="c", subcore_axis_name="s")
    kfn = functools.partial(
        pl.kernel,
        mesh=mesh,
        out_type=[
            jax.ShapeDtypeStruct((NSC,), jnp.float32),
            jax.ShapeDtypeStruct((NSC,), jnp.float32),
        ],
        scratch_types=[
            pltpu.VMEM((C * SCQ,), jnp.float32),
            pltpu.VMEM((SCQ,), jnp.int32),
            pltpu.VMEM((SCQ,), jnp.float32),
            pltpu.VMEM((SCQ,), jnp.float32),
        ],
    )(_sc_body)
    return kfn(score_tail, target_tail)


def _ce_pg_kernel(score_ref, target_ref, ce_ref, pg_ref):
    ce, pg = _ce_pg(score_ref[0, :, 0], target_ref[0, 0, 0])
    ce_ref[0, 0, 0] = ce
    pg_ref[0, 0, 0] = pg


SEL_ROWS = 64
SEL_CHUNK = 8
SEL_ITERS = 19
SEL_W = N // SEL_ROWS


def _select_kernel(pg_ref, ce_ref, out_ref):
    nchunks = SEL_ROWS // SEL_CHUNK

    def count_le(v):
        def body(j, acc):
            blk = lax.bitcast_convert_type(
                pg_ref[pl.ds(j * SEL_CHUNK, SEL_CHUNK), :], jnp.int32)
            return acc + (blk <= v).astype(jnp.int32)
        acc = lax.fori_loop(
            0, nchunks, body, jnp.zeros((SEL_CHUNK, SEL_W), jnp.int32))
        return jnp.sum(acc)

    c7 = count_le(jnp.int32(THRESH_BITS - 1))

    def bisect(_, carry):
        lo, hi = carry
        mid = lo + (hi - lo) // 2
        big = count_le(mid) >= (KK + 1)
        new_lo = jnp.where(big, lo, mid)
        new_hi = jnp.where(big, mid, hi)
        done = (hi - lo) <= 1
        return (jnp.where(done, lo, new_lo), jnp.where(done, hi, new_hi))

    lo0 = jnp.int32(THRESH_BITS - 1)
    hi0 = jnp.int32(ONE_BITS)
    _, kth_bits = lax.fori_loop(0, SEL_ITERS, bisect, (lo0, hi0))

    thr_bits = jnp.where(c7 >= (KK + 1), jnp.int32(THRESH_BITS), kth_bits)

    def final_body(j, carry):
        s_acc, c_acc = carry
        sl = pl.ds(j * SEL_CHUNK, SEL_CHUNK)
        blk = lax.bitcast_convert_type(pg_ref[sl, :], jnp.int32)
        keep = (blk < thr_bits).astype(jnp.float32)
        return (s_acc + ce_ref[sl, :] * keep, c_acc + keep)

    z = jnp.zeros((SEL_CHUNK, SEL_W), jnp.float32)
    s_acc, c_acc = lax.fori_loop(0, nchunks, final_body, (z, z))
    loss = jnp.sum(s_acc) / jnp.maximum(jnp.sum(c_acc), jnp.float32(1.0))
    out_ref[...] = loss[None, None]


def _dense_specs():
    return dict(
        in_specs=[
            pl.BlockSpec((1, C, 1, 8, TL), lambda b, g: (b, 0, g, 0, 0)),
            pl.BlockSpec((1, 1, 1, 8, TL), lambda b, g: (b, 0, g, 0, 0)),
        ],
    )


@jax.jit
def kernel(score, target):
    s5 = score.reshape(B, C, NG, 8, TL)
    t5 = target.reshape(B, 1, NG, 8, TL)

    score_tail = score[SPLIT:].transpose(1, 0, 2, 3).reshape(C * NSC)
    target_tail = target[SPLIT:].reshape(NSC)
    s_sc, stm_sc = _sc_stage(score_tail, target_tail)

    s7a, c7a = pl.pallas_call(
        functools.partial(_fused_kernel, SPLIT),
        grid=(SPLIT, NG),
        out_specs=[
            pl.BlockSpec((1, 1), lambda b, g: (0, 0)),
            pl.BlockSpec((1, 1), lambda b, g: (0, 0)),
        ],
        out_shape=[
            jax.ShapeDtypeStruct((1, 1), jnp.float32),
            jax.ShapeDtypeStruct((1, 1), jnp.float32),
        ],
        scratch_shapes=[
            pltpu.VMEM((8, TL), jnp.float32),
            pltpu.VMEM((8, TL), jnp.float32),
        ],
        **_dense_specs(),
    )(s5[:SPLIT], t5[:SPLIT])

    ngf = NSC // (8 * TL)
    s7b, c7b = pl.pallas_call(
        _fold_kernel,
        grid=(ngf,),
        in_specs=[
            pl.BlockSpec((1, 8, TL), lambda g: (g, 0, 0)),
            pl.BlockSpec((1, 8, TL), lambda g: (g, 0, 0)),
        ],
        out_specs=[
            pl.BlockSpec((1, 1), lambda g: (0, 0)),
            pl.BlockSpec((1, 1), lambda g: (0, 0)),
        ],
        out_shape=[
            jax.ShapeDtypeStruct((1, 1), jnp.float32),
            jax.ShapeDtypeStruct((1, 1), jnp.float32),
        ],
        scratch_shapes=[
            pltpu.VMEM((8, TL), jnp.float32),
            pltpu.VMEM((8, TL), jnp.float32),
        ],
    )(s_sc.reshape(ngf, 8, TL), stm_sc.reshape(ngf, 8, TL))

    s7 = s7a[0, 0] + s7b[0, 0]
    c7 = c7a[0, 0] + c7b[0, 0]

    def common_case():
        return s7 / jnp.maximum(c7, jnp.float32(1.0))

    def rare_case():
        ce, pg = pl.pallas_call(
            _ce_pg_kernel,
            grid=(B, NG),
            out_specs=[
                pl.BlockSpec((1, 1, 1, 8, TL), lambda b, g: (b, 0, g, 0, 0)),
                pl.BlockSpec((1, 1, 1, 8, TL), lambda b, g: (b, 0, g, 0, 0)),
            ],
            out_shape=[
                jax.ShapeDtypeStruct((B, 1, NG, 8, TL), jnp.float32),
                jax.ShapeDtypeStruct((B, 1, NG, 8, TL), jnp.float32),
            ],
            **_dense_specs(),
        )(s5, t5)
        out = pl.pallas_call(
            _select_kernel,
            out_shape=jax.ShapeDtypeStruct((1, 1), jnp.float32),
        )(pg.reshape(SEL_ROWS, SEL_W), ce.reshape(SEL_ROWS, SEL_W))
        return out[0, 0]

    return lax.cond(c7 >= jnp.float32(KK + 1), common_case, rare_case)


# speedup vs baseline: 4.5791x; 4.5791x over previous
import jax
import jax.numpy as jnp
import numpy as np
from jax import lax
from jax.experimental import pallas as pl
from jax.experimental.pallas import tpu as pltpu

B = 8
C = 19
H = 512
W = 512
P = H * W
N = B * P
KK = 100000
THRESH = 0.7
THRESH_BITS = int(np.float32(THRESH).view(np.int32))
ONE_BITS = int(np.float32(1.0).view(np.int32))

RB = 64
NG = H // RB


def _ce_pg(x, t):
    m = jnp.max(x, axis=0)
    e = jnp.exp(x - m[None])
    s = jnp.sum(e, axis=0)
    iota = lax.broadcasted_iota(jnp.int32, (C, RB, W), 0)
    onehot = (iota == t[None]).astype(jnp.float32)
    st = jnp.sum(x * onehot, axis=0)
    ce = (m + jnp.log(s)) - st
    pg = jnp.exp(st - m) / s
    return ce, pg


def _fused_kernel(score_ref, target_ref, s7_ref, c7_ref, acc_s, acc_c):
    b = pl.program_id(0)
    g = pl.program_id(1)

    @pl.when((b == 0) & (g == 0))
    def _init():
        acc_s[...] = jnp.zeros((RB, W), jnp.float32)
        acc_c[...] = jnp.zeros((RB, W), jnp.float32)

    ce, pg = _ce_pg(score_ref[0], target_ref[0])
    keep = (pg < THRESH).astype(jnp.float32)
    acc_s[...] += ce * keep
    acc_c[...] += keep

    @pl.when((b == B - 1) & (g == NG - 1))
    def _finish():
        s7_ref[...] = jnp.sum(acc_s[...])[None, None]
        c7_ref[...] = jnp.sum(acc_c[...])[None, None]


def _ce_pg_kernel(score_ref, target_ref, ce_ref, pg_ref):
    ce, pg = _ce_pg(score_ref[0], target_ref[0])
    ce_ref[0] = ce
    pg_ref[0] = pg


SEL_ROWS = 64
SEL_CHUNK = 8
SEL_ITERS = 19
SEL_W = N // SEL_ROWS


def _select_kernel(pg_ref, ce_ref, out_ref):
    nchunks = SEL_ROWS // SEL_CHUNK

    def count_le(v):
        def body(j, acc):
            blk = lax.bitcast_convert_type(
                pg_ref[pl.ds(j * SEL_CHUNK, SEL_CHUNK), :], jnp.int32)
            return acc + (blk <= v).astype(jnp.int32)
        acc = lax.fori_loop(
            0, nchunks, body, jnp.zeros((SEL_CHUNK, SEL_W), jnp.int32))
        return jnp.sum(acc)

    c7 = count_le(jnp.int32(THRESH_BITS - 1))

    def bisect(_, carry):
        lo, hi = carry
        mid = lo + (hi - lo) // 2
        big = count_le(mid) >= (KK + 1)
        new_lo = jnp.where(big, lo, mid)
        new_hi = jnp.where(big, mid, hi)
        done = (hi - lo) <= 1
        return (jnp.where(done, lo, new_lo), jnp.where(done, hi, new_hi))

    lo0 = jnp.int32(THRESH_BITS - 1)
    hi0 = jnp.int32(ONE_BITS)
    _, kth_bits = lax.fori_loop(0, SEL_ITERS, bisect, (lo0, hi0))

    thr_bits = jnp.where(c7 >= (KK + 1), jnp.int32(THRESH_BITS), kth_bits)

    def final_body(j, carry):
        s_acc, c_acc = carry
        sl = pl.ds(j * SEL_CHUNK, SEL_CHUNK)
        blk = lax.bitcast_convert_type(pg_ref[sl, :], jnp.int32)
        keep = (blk < thr_bits).astype(jnp.float32)
        return (s_acc + ce_ref[sl, :] * keep, c_acc + keep)

    z = jnp.zeros((SEL_CHUNK, SEL_W), jnp.float32)
    s_acc, c_acc = lax.fori_loop(0, nchunks, final_body, (z, z))
    loss = jnp.sum(s_acc) / jnp.maximum(jnp.sum(c_acc), jnp.float32(1.0))
    out_ref[...] = loss[None, None]


@jax.jit
def kernel(score, target):
    grid = (B, NG)
    in_specs = [
        pl.BlockSpec((1, C, RB, W), lambda b, g: (b, 0, g, 0)),
        pl.BlockSpec((1, RB, W), lambda b, g: (b, g, 0)),
    ]

    s7, c7 = pl.pallas_call(
        _fused_kernel,
        grid=grid,
        in_specs=in_specs,
        out_specs=[
            pl.BlockSpec((1, 1), lambda b, g: (0, 0)),
            pl.BlockSpec((1, 1), lambda b, g: (0, 0)),
        ],
        out_shape=[
            jax.ShapeDtypeStruct((1, 1), jnp.float32),
            jax.ShapeDtypeStruct((1, 1), jnp.float32),
        ],
        scratch_shapes=[
            pltpu.VMEM((RB, W), jnp.float32),
            pltpu.VMEM((RB, W), jnp.float32),
        ],
    )(score, target)
    s7 = s7[0, 0]
    c7 = c7[0, 0]

    def common_case():
        return s7 / jnp.maximum(c7, jnp.float32(1.0))

    def rare_case():
        ce, pg = pl.pallas_call(
            _ce_pg_kernel,
            grid=grid,
            in_specs=in_specs,
            out_specs=[
                pl.BlockSpec((1, RB, W), lambda b, g: (b, g, 0)),
                pl.BlockSpec((1, RB, W), lambda b, g: (b, g, 0)),
            ],
            out_shape=[
                jax.ShapeDtypeStruct((B, H, W), jnp.float32),
                jax.ShapeDtypeStruct((B, H, W), jnp.float32),
            ],
        )(score, target)
        out = pl.pallas_call(
            _select_kernel,
            out_shape=jax.ShapeDtypeStruct((1, 1), jnp.float32),
        )(pg.reshape(SEL_ROWS, SEL_W), ce.reshape(SEL_ROWS, SEL_W))
        return out[0, 0]

    return lax.cond(c7 >= jnp.float32(KK + 1), common_case, rare_case)


# speedup vs baseline: 5.6580x; 1.2356x over previous
import jax
import jax.numpy as jnp
import numpy as np
from jax import lax
from jax.experimental import pallas as pl
from jax.experimental.pallas import tpu as pltpu

B = 8
C = 19
H = 512
W = 512
P = H * W
N = B * P
KK = 100000
THRESH = 0.7
THRESH_BITS = int(np.float32(THRESH).view(np.int32))
ONE_BITS = int(np.float32(1.0).view(np.int32))

RB = 128
NG = H // RB


def _ce_pg(x, t):
    m = jnp.max(x, axis=0)
    e = jnp.exp(x - m[None])
    s = jnp.sum(e, axis=0)
    iota = lax.broadcasted_iota(jnp.int32, (C, RB, W), 0)
    onehot = (iota == t[None]).astype(jnp.float32)
    st = jnp.sum(x * onehot, axis=0)
    ce = (m + jnp.log(s)) - st
    pg = jnp.exp(st - m) / s
    return ce, pg


def _fused_kernel(score_ref, target_ref, s7_ref, c7_ref, acc_s, acc_c):
    b = pl.program_id(0)
    g = pl.program_id(1)

    @pl.when((b == 0) & (g == 0))
    def _init():
        acc_s[...] = jnp.zeros((RB, W), jnp.float32)
        acc_c[...] = jnp.zeros((RB, W), jnp.float32)

    ce, pg = _ce_pg(score_ref[0], target_ref[0])
    keep = (pg < THRESH).astype(jnp.float32)
    acc_s[...] += ce * keep
    acc_c[...] += keep

    @pl.when((b == B - 1) & (g == NG - 1))
    def _finish():
        s7_ref[...] = jnp.sum(acc_s[...])[None, None]
        c7_ref[...] = jnp.sum(acc_c[...])[None, None]


def _ce_pg_kernel(score_ref, target_ref, ce_ref, pg_ref):
    ce, pg = _ce_pg(score_ref[0], target_ref[0])
    ce_ref[0] = ce
    pg_ref[0] = pg


SEL_ROWS = 64
SEL_CHUNK = 8
SEL_ITERS = 19
SEL_W = N // SEL_ROWS


def _select_kernel(pg_ref, ce_ref, out_ref):
    nchunks = SEL_ROWS // SEL_CHUNK

    def count_le(v):
        def body(j, acc):
            blk = lax.bitcast_convert_type(
                pg_ref[pl.ds(j * SEL_CHUNK, SEL_CHUNK), :], jnp.int32)
            return acc + (blk <= v).astype(jnp.int32)
        acc = lax.fori_loop(
            0, nchunks, body, jnp.zeros((SEL_CHUNK, SEL_W), jnp.int32))
        return jnp.sum(acc)

    c7 = count_le(jnp.int32(THRESH_BITS - 1))

    def bisect(_, carry):
        lo, hi = carry
        mid = lo + (hi - lo) // 2
        big = count_le(mid) >= (KK + 1)
        new_lo = jnp.where(big, lo, mid)
        new_hi = jnp.where(big, mid, hi)
        done = (hi - lo) <= 1
        return (jnp.where(done, lo, new_lo), jnp.where(done, hi, new_hi))

    lo0 = jnp.int32(THRESH_BITS - 1)
    hi0 = jnp.int32(ONE_BITS)
    _, kth_bits = lax.fori_loop(0, SEL_ITERS, bisect, (lo0, hi0))

    thr_bits = jnp.where(c7 >= (KK + 1), jnp.int32(THRESH_BITS), kth_bits)

    def final_body(j, carry):
        s_acc, c_acc = carry
        sl = pl.ds(j * SEL_CHUNK, SEL_CHUNK)
        blk = lax.bitcast_convert_type(pg_ref[sl, :], jnp.int32)
        keep = (blk < thr_bits).astype(jnp.float32)
        return (s_acc + ce_ref[sl, :] * keep, c_acc + keep)

    z = jnp.zeros((SEL_CHUNK, SEL_W), jnp.float32)
    s_acc, c_acc = lax.fori_loop(0, nchunks, final_body, (z, z))
    loss = jnp.sum(s_acc) / jnp.maximum(jnp.sum(c_acc), jnp.float32(1.0))
    out_ref[...] = loss[None, None]


@jax.jit
def kernel(score, target):
    grid = (B, NG)
    in_specs = [
        pl.BlockSpec((1, C, RB, W), lambda b, g: (b, 0, g, 0)),
        pl.BlockSpec((1, RB, W), lambda b, g: (b, g, 0)),
    ]

    s7, c7 = pl.pallas_call(
        _fused_kernel,
        grid=grid,
        in_specs=in_specs,
        out_specs=[
            pl.BlockSpec((1, 1), lambda b, g: (0, 0)),
            pl.BlockSpec((1, 1), lambda b, g: (0, 0)),
        ],
        out_shape=[
            jax.ShapeDtypeStruct((1, 1), jnp.float32),
            jax.ShapeDtypeStruct((1, 1), jnp.float32),
        ],
        scratch_shapes=[
            pltpu.VMEM((RB, W), jnp.float32),
            pltpu.VMEM((RB, W), jnp.float32),
        ],
    )(score, target)
    s7 = s7[0, 0]
    c7 = c7[0, 0]

    def common_case():
        return s7 / jnp.maximum(c7, jnp.float32(1.0))

    def rare_case():
        ce, pg = pl.pallas_call(
            _ce_pg_kernel,
            grid=grid,
            in_specs=in_specs,
            out_specs=[
                pl.BlockSpec((1, RB, W), lambda b, g: (b, g, 0)),
                pl.BlockSpec((1, RB, W), lambda b, g: (b, g, 0)),
            ],
            out_shape=[
                jax.ShapeDtypeStruct((B, H, W), jnp.float32),
                jax.ShapeDtypeStruct((B, H, W), jnp.float32),
            ],
        )(score, target)
        out = pl.pallas_call(
            _select_kernel,
            out_shape=jax.ShapeDtypeStruct((1, 1), jnp.float32),
        )(pg.reshape(SEL_ROWS, SEL_W), ce.reshape(SEL_ROWS, SEL_W))
        return out[0, 0]

    return lax.cond(c7 >= jnp.float32(KK + 1), common_case, rare_case)


# speedup vs baseline: 6.2929x; 1.1122x over previous
import jax
import jax.numpy as jnp
import numpy as np
from jax import lax
from jax.experimental import pallas as pl
from jax.experimental.pallas import tpu as pltpu

B = 8
C = 19
H = 512
W = 512
P = H * W
N = B * P
KK = 100000
THRESH = 0.7
THRESH_BITS = int(np.float32(THRESH).view(np.int32))
ONE_BITS = int(np.float32(1.0).view(np.int32))

RB = 256
NG = H // RB


def _ce_pg(x, t):
    m = jnp.max(x, axis=0)
    e = jnp.exp(x - m[None])
    s = jnp.sum(e, axis=0)
    iota = lax.broadcasted_iota(jnp.int32, (C, RB, W), 0)
    onehot = (iota == t[None]).astype(jnp.float32)
    st = jnp.sum(x * onehot, axis=0)
    ce = (m + jnp.log(s)) - st
    pg = jnp.exp(st - m) / s
    return ce, pg


def _fused_kernel(score_ref, target_ref, s7_ref, c7_ref, acc_s, acc_c):
    b = pl.program_id(0)
    g = pl.program_id(1)

    @pl.when((b == 0) & (g == 0))
    def _init():
        acc_s[...] = jnp.zeros((RB, W), jnp.float32)
        acc_c[...] = jnp.zeros((RB, W), jnp.float32)

    ce, pg = _ce_pg(score_ref[0], target_ref[0])
    keep = (pg < THRESH).astype(jnp.float32)
    acc_s[...] += ce * keep
    acc_c[...] += keep

    @pl.when((b == B - 1) & (g == NG - 1))
    def _finish():
        s7_ref[...] = jnp.sum(acc_s[...])[None, None]
        c7_ref[...] = jnp.sum(acc_c[...])[None, None]


def _ce_pg_kernel(score_ref, target_ref, ce_ref, pg_ref):
    ce, pg = _ce_pg(score_ref[0], target_ref[0])
    ce_ref[0] = ce
    pg_ref[0] = pg


SEL_ROWS = 64
SEL_CHUNK = 8
SEL_ITERS = 19
SEL_W = N // SEL_ROWS


def _select_kernel(pg_ref, ce_ref, out_ref):
    nchunks = SEL_ROWS // SEL_CHUNK

    def count_le(v):
        def body(j, acc):
            blk = lax.bitcast_convert_type(
                pg_ref[pl.ds(j * SEL_CHUNK, SEL_CHUNK), :], jnp.int32)
            return acc + (blk <= v).astype(jnp.int32)
        acc = lax.fori_loop(
            0, nchunks, body, jnp.zeros((SEL_CHUNK, SEL_W), jnp.int32))
        return jnp.sum(acc)

    c7 = count_le(jnp.int32(THRESH_BITS - 1))

    def bisect(_, carry):
        lo, hi = carry
        mid = lo + (hi - lo) // 2
        big = count_le(mid) >= (KK + 1)
        new_lo = jnp.where(big, lo, mid)
        new_hi = jnp.where(big, mid, hi)
        done = (hi - lo) <= 1
        return (jnp.where(done, lo, new_lo), jnp.where(done, hi, new_hi))

    lo0 = jnp.int32(THRESH_BITS - 1)
    hi0 = jnp.int32(ONE_BITS)
    _, kth_bits = lax.fori_loop(0, SEL_ITERS, bisect, (lo0, hi0))

    thr_bits = jnp.where(c7 >= (KK + 1), jnp.int32(THRESH_BITS), kth_bits)

    def final_body(j, carry):
        s_acc, c_acc = carry
        sl = pl.ds(j * SEL_CHUNK, SEL_CHUNK)
        blk = lax.bitcast_convert_type(pg_ref[sl, :], jnp.int32)
        keep = (blk < thr_bits).astype(jnp.float32)
        return (s_acc + ce_ref[sl, :] * keep, c_acc + keep)

    z = jnp.zeros((SEL_CHUNK, SEL_W), jnp.float32)
    s_acc, c_acc = lax.fori_loop(0, nchunks, final_body, (z, z))
    loss = jnp.sum(s_acc) / jnp.maximum(jnp.sum(c_acc), jnp.float32(1.0))
    out_ref[...] = loss[None, None]


@jax.jit
def kernel(score, target):
    grid = (B, NG)
    in_specs = [
        pl.BlockSpec((1, C, RB, W), lambda b, g: (b, 0, g, 0)),
        pl.BlockSpec((1, RB, W), lambda b, g: (b, g, 0)),
    ]

    s7, c7 = pl.pallas_call(
        _fused_kernel,
        grid=grid,
        in_specs=in_specs,
        out_specs=[
            pl.BlockSpec((1, 1), lambda b, g: (0, 0)),
            pl.BlockSpec((1, 1), lambda b, g: (0, 0)),
        ],
        out_shape=[
            jax.ShapeDtypeStruct((1, 1), jnp.float32),
            jax.ShapeDtypeStruct((1, 1), jnp.float32),
        ],
        scratch_shapes=[
            pltpu.VMEM((RB, W), jnp.float32),
            pltpu.VMEM((RB, W), jnp.float32),
        ],
    )(score, target)
    s7 = s7[0, 0]
    c7 = c7[0, 0]

    def common_case():
        return s7 / jnp.maximum(c7, jnp.float32(1.0))

    def rare_case():
        ce, pg = pl.pallas_call(
            _ce_pg_kernel,
            grid=grid,
            in_specs=in_specs,
            out_specs=[
                pl.BlockSpec((1, RB, W), lambda b, g: (b, g, 0)),
                pl.BlockSpec((1, RB, W), lambda b, g: (b, g, 0)),
            ],
            out_shape=[
                jax.ShapeDtypeStruct((B, H, W), jnp.float32),
                jax.ShapeDtypeStruct((B, H, W), jnp.float32),
            ],
        )(score, target)
        out = pl.pallas_call(
            _select_kernel,
            out_shape=jax.ShapeDtypeStruct((1, 1), jnp.float32),
        )(pg.reshape(SEL_ROWS, SEL_W), ce.reshape(SEL_ROWS, SEL_W))
        return out[0, 0]

    return lax.cond(c7 >= jnp.float32(KK + 1), common_case, rare_case)


# speedup vs baseline: 6.4004x; 1.0171x over previous
import jax
import jax.numpy as jnp
import numpy as np
from jax import lax
from jax.experimental import pallas as pl
from jax.experimental.pallas import tpu as pltpu

B = 8
C = 19
H = 512
W = 512
P = H * W
N = B * P
KK = 100000
THRESH = 0.7
THRESH_BITS = int(np.float32(THRESH).view(np.int32))
ONE_BITS = int(np.float32(1.0).view(np.int32))

RB = 512
NG = H // RB


def _ce_pg(x, t):
    m = jnp.max(x, axis=0)
    e = jnp.exp(x - m[None])
    s = jnp.sum(e, axis=0)
    iota = lax.broadcasted_iota(jnp.int32, (C, RB, W), 0)
    onehot = (iota == t[None]).astype(jnp.float32)
    st = jnp.sum(x * onehot, axis=0)
    ce = (m + jnp.log(s)) - st
    pg = jnp.exp(st - m) / s
    return ce, pg


def _fused_kernel(score_ref, target_ref, s7_ref, c7_ref, acc_s, acc_c):
    b = pl.program_id(0)
    g = pl.program_id(1)

    @pl.when((b == 0) & (g == 0))
    def _init():
        acc_s[...] = jnp.zeros((RB, W), jnp.float32)
        acc_c[...] = jnp.zeros((RB, W), jnp.float32)

    ce, pg = _ce_pg(score_ref[0], target_ref[0])
    keep = (pg < THRESH).astype(jnp.float32)
    acc_s[...] += ce * keep
    acc_c[...] += keep

    @pl.when((b == B - 1) & (g == NG - 1))
    def _finish():
        s7_ref[...] = jnp.sum(acc_s[...])[None, None]
        c7_ref[...] = jnp.sum(acc_c[...])[None, None]


def _ce_pg_kernel(score_ref, target_ref, ce_ref, pg_ref):
    ce, pg = _ce_pg(score_ref[0], target_ref[0])
    ce_ref[0] = ce
    pg_ref[0] = pg


SEL_ROWS = 64
SEL_CHUNK = 8
SEL_ITERS = 19
SEL_W = N // SEL_ROWS


def _select_kernel(pg_ref, ce_ref, out_ref):
    nchunks = SEL_ROWS // SEL_CHUNK

    def count_le(v):
        def body(j, acc):
            blk = lax.bitcast_convert_type(
                pg_ref[pl.ds(j * SEL_CHUNK, SEL_CHUNK), :], jnp.int32)
            return acc + (blk <= v).astype(jnp.int32)
        acc = lax.fori_loop(
            0, nchunks, body, jnp.zeros((SEL_CHUNK, SEL_W), jnp.int32))
        return jnp.sum(acc)

    c7 = count_le(jnp.int32(THRESH_BITS - 1))

    def bisect(_, carry):
        lo, hi = carry
        mid = lo + (hi - lo) // 2
        big = count_le(mid) >= (KK + 1)
        new_lo = jnp.where(big, lo, mid)
        new_hi = jnp.where(big, mid, hi)
        done = (hi - lo) <= 1
        return (jnp.where(done, lo, new_lo), jnp.where(done, hi, new_hi))

    lo0 = jnp.int32(THRESH_BITS - 1)
    hi0 = jnp.int32(ONE_BITS)
    _, kth_bits = lax.fori_loop(0, SEL_ITERS, bisect, (lo0, hi0))

    thr_bits = jnp.where(c7 >= (KK + 1), jnp.int32(THRESH_BITS), kth_bits)

    def final_body(j, carry):
        s_acc, c_acc = carry
        sl = pl.ds(j * SEL_CHUNK, SEL_CHUNK)
        blk = lax.bitcast_convert_type(pg_ref[sl, :], jnp.int32)
        keep = (blk < thr_bits).astype(jnp.float32)
        return (s_acc + ce_ref[sl, :] * keep, c_acc + keep)

    z = jnp.zeros((SEL_CHUNK, SEL_W), jnp.float32)
    s_acc, c_acc = lax.fori_loop(0, nchunks, final_body, (z, z))
    loss = jnp.sum(s_acc) / jnp.maximum(jnp.sum(c_acc), jnp.float32(1.0))
    out_ref[...] = loss[None, None]


@jax.jit
def kernel(score, target):
    grid = (B, NG)
    in_specs = [
        pl.BlockSpec((1, C, RB, W), lambda b, g: (b, 0, g, 0)),
        pl.BlockSpec((1, RB, W), lambda b, g: (b, g, 0)),
    ]

    s7, c7 = pl.pallas_call(
        _fused_kernel,
        grid=grid,
        in_specs=in_specs,
        out_specs=[
            pl.BlockSpec((1, 1), lambda b, g: (0, 0)),
            pl.BlockSpec((1, 1), lambda b, g: (0, 0)),
        ],
        out_shape=[
            jax.ShapeDtypeStruct((1, 1), jnp.float32),
            jax.ShapeDtypeStruct((1, 1), jnp.float32),
        ],
        scratch_shapes=[
            pltpu.VMEM((RB, W), jnp.float32),
            pltpu.VMEM((RB, W), jnp.float32),
        ],
    )(score, target)
    s7 = s7[0, 0]
    c7 = c7[0, 0]

    def common_case():
        return s7 / jnp.maximum(c7, jnp.float32(1.0))

    def rare_case():
        ce, pg = pl.pallas_call(
            _ce_pg_kernel,
            grid=grid,
            in_specs=in_specs,
            out_specs=[
                pl.BlockSpec((1, RB, W), lambda b, g: (b, g, 0)),
                pl.BlockSpec((1, RB, W), lambda b, g: (b, g, 0)),
            ],
            out_shape=[
                jax.ShapeDtypeStruct((B, H, W), jnp.float32),
                jax.ShapeDtypeStruct((B, H, W), jnp.float32),
            ],
        )(score, target)
        out = pl.pallas_call(
            _select_kernel,
            out_shape=jax.ShapeDtypeStruct((1, 1), jnp.float32),
        )(pg.reshape(SEL_ROWS, SEL_W), ce.reshape(SEL_ROWS, SEL_W))
        return out[0, 0]

    return lax.cond(c7 >= jnp.float32(KK + 1), common_case, rare_case)
